# Initial kernel scaffold; baseline (speedup 1.0000x reference)
import jax
import jax.numpy as jnp
from jax.experimental import pallas as pl


def kernel(x, emb):
    raise NotImplementedError("write your pallas kernel here")


# speedup vs baseline: 2.5991x; 2.5991x over previous
import jax
import jax.numpy as jnp
from jax.experimental import pallas as pl

_SCALE = 2048 ** -0.5
_BLK = 1024


def _scale_copy(emb_ref, o_ref):
    o_ref[...] = emb_ref[...] * _SCALE


def kernel(x, emb):
    s, d = emb.shape
    out = pl.pallas_call(
        _scale_copy,
        grid=(s // _BLK,),
        in_specs=[pl.BlockSpec((_BLK, d), lambda i: (i, 0))],
        out_specs=pl.BlockSpec((_BLK, d), lambda i: (i, 0)),
        out_shape=jax.ShapeDtypeStruct((s, d), emb.dtype),
    )(emb)
    return out[None]
